# baseline (speedup 1.0000x reference)
import functools

import jax
import jax.numpy as jnp
from jax import lax
from jax.experimental import pallas as pl
from jax.experimental.pallas import tpu as pltpu
from jax.experimental.pallas import tpu_sc as plsc

N_SEG = 2048
L = 16


@functools.lru_cache(maxsize=None)
def _build_sc_kernel(B, F, P, CPW, CHUNK):
    NCHUNK = P // CHUNK
    GROUPS = CHUNK // L
    WPB = F // CPW
    assert B * WPB == 32, "mapping assumes 32 SC subcores"

    mesh = plsc.VectorSubcoreMesh(core_axis_name="c", subcore_axis_name="s")

    @functools.partial(
        pl.kernel,
        mesh=mesh,
        compiler_params=pltpu.CompilerParams(needs_layout_passes=False),
        out_type=[
            jax.ShapeDtypeStruct((B * F * N_SEG,), jnp.float32),
            jax.ShapeDtypeStruct((B * N_SEG,), jnp.float32),
            jax.ShapeDtypeStruct((B * 2 * N_SEG,), jnp.float32),
        ],
        scratch_types=[
            pltpu.VMEM((2, CHUNK), jnp.int32),
            pltpu.VMEM((2, CPW, CHUNK), jnp.float32),
            pltpu.VMEM((CPW * N_SEG,), jnp.float32),
            pltpu.VMEM((N_SEG,), jnp.float32),
            pltpu.VMEM((N_SEG,), jnp.float32),
            pltpu.VMEM((N_SEG,), jnp.float32),
            pltpu.SemaphoreType.DMA,
            pltpu.SemaphoreType.DMA,
        ],
    )
    def sp_kernel(feat_hbm, seg_hbm, tok_hbm, cnt_hbm, cent_hbm,
                  seg_v, feat_v, acc_v, cnt_v, cent_v, recip_v,
                  sem0, sem1):
        cid = lax.axis_index("c")
        sid = lax.axis_index("s")
        batch = cid * (B // 2) + sid // WPB
        wib = sid % WPB
        c0 = wib * CPW

        iota = lax.iota(jnp.int32, L)
        zeros = jnp.zeros((L,), jnp.float32)
        ones = jnp.ones((L,), jnp.float32)
        col_base = [jnp.full((L,), cc * N_SEG, jnp.int32) for cc in range(CPW)]

        def zero_small(i, _):
            cnt_v[pl.ds(i * L, L)] = zeros
            cent_v[pl.ds(i * L, L)] = zeros
            return 0
        lax.fori_loop(0, N_SEG // L, zero_small, 0)

        @plsc.parallel_loop(0, (CPW * N_SEG) // L, unroll=8)
        def zero_acc(i):
            acc_v[pl.ds(i * L, L)] = zeros

        feat_base = (batch * F + c0) * P
        sems = [sem0, sem1]

        def copies(g, sl):
            base_px = g * CHUNK
            cps = [pltpu.make_async_copy(
                seg_hbm.at[pl.ds(batch * P + base_px, CHUNK)],
                seg_v.at[sl], sems[sl])]
            for cc in range(CPW):
                cps.append(pltpu.make_async_copy(
                    feat_hbm.at[pl.ds(feat_base + cc * P + base_px, CHUNK)],
                    feat_v.at[sl, cc], sems[sl]))
            return cps

        def issue(g, sl):
            for cp in copies(g, sl):
                cp.start()

        def drain(g, sl):
            for cp in copies(g, sl):
                cp.wait()

        issue(0, 0)
        issue(1, 1)

        def process(g, sl, base_px):
            @plsc.parallel_loop(0, GROUPS, unroll=8)
            def grp(i):
                off = i * L
                seg16 = seg_v[sl, pl.ds(off, L)]
                for cc in range(CPW):
                    val = feat_v[sl, cc, pl.ds(off, L)]
                    plsc.addupdate_scatter(acc_v, [seg16 + col_base[cc]], val)
                plsc.addupdate_scatter(cnt_v, [seg16], ones)

            @pl.when(jnp.logical_or(wib == 1, wib == 2))
            def _():
                is_x = jnp.full((L,), wib) == 1

                @plsc.parallel_loop(0, GROUPS, unroll=4)
                def grpc(i):
                    off = i * L
                    seg16 = seg_v[sl, pl.ds(off, L)]
                    p = base_px + off + iota
                    x = (p & 511).astype(jnp.float32)
                    y = (p >> 9).astype(jnp.float32)
                    v = jnp.where(is_x, x, y) * (1.0 / 511.0)
                    plsc.addupdate_scatter(cent_v, [seg16], v)

        def outer(gg, _):
            for sl in range(2):
                g = gg * 2 + sl
                drain(g, sl)
                process(g, sl, g * CHUNK)

                @pl.when(g + 2 < NCHUNK)
                def _():
                    issue(g + 2, sl)
            return 0
        lax.fori_loop(0, NCHUNK // 2, outer, 0)

        def rec(i, _):
            c16 = cnt_v[pl.ds(i * L, L)]
            recip_v[pl.ds(i * L, L)] = 1.0 / jnp.maximum(c16, 1.0)
            return 0
        lax.fori_loop(0, N_SEG // L, rec, 0)

        @plsc.parallel_loop(0, (CPW * N_SEG) // L, unroll=8)
        def div_acc(i):
            s = pl.ds(i * L, L)
            r = pl.ds((i * L) & (N_SEG - 1), L)
            acc_v[s] = acc_v[s] * recip_v[r]

        pltpu.sync_copy(
            acc_v, tok_hbm.at[pl.ds((batch * F + c0) * N_SEG, CPW * N_SEG)])

        @pl.when(wib == 0)
        def _():
            pltpu.sync_copy(cnt_v, cnt_hbm.at[pl.ds(batch * N_SEG, N_SEG)])

        @pl.when(jnp.logical_or(wib == 1, wib == 2))
        def _():
            def div_cent(i, _):
                s = pl.ds(i * L, L)
                cent_v[s] = cent_v[s] * recip_v[s]
                return 0
            lax.fori_loop(0, N_SEG // L, div_cent, 0)
            pltpu.sync_copy(
                cent_v,
                cent_hbm.at[pl.ds((batch * 2 + (wib - 1)) * N_SEG, N_SEG)])

    return sp_kernel


def kernel(images, features, segment_map):
    B, F, H, W = features.shape
    P = H * W
    feats = features.reshape(B * F * P)
    segs = segment_map.reshape(B * P)
    tok1d, cnt1d, cent1d = _build_sc_kernel(B, F, P, 12, 2048)(feats, segs)
    tokens = tok1d.reshape(B, F, N_SEG).transpose(0, 2, 1)
    counts = cnt1d.reshape(B, N_SEG)
    centroids = cent1d.reshape(B, 2, N_SEG).transpose(0, 2, 1)
    attention_mask = counts > 0
    return (tokens, segment_map, attention_mask, centroids)


# speedup vs baseline: 1.1535x; 1.1535x over previous
import functools

import jax
import jax.numpy as jnp
from jax import lax
from jax.experimental import pallas as pl
from jax.experimental.pallas import tpu as pltpu
from jax.experimental.pallas import tpu_sc as plsc

N_SEG = 2048
L = 16


@functools.lru_cache(maxsize=None)
def _build_sc_kernel(B, F, P, CPW, CHUNK):
    NCHUNK = P // CHUNK
    GROUPS = CHUNK // L
    WPB = F // CPW
    assert B * WPB == 32, "mapping assumes 32 SC subcores"

    mesh = plsc.VectorSubcoreMesh(core_axis_name="c", subcore_axis_name="s")

    @functools.partial(
        pl.kernel,
        mesh=mesh,
        compiler_params=pltpu.CompilerParams(needs_layout_passes=False),
        out_type=[
            jax.ShapeDtypeStruct((B * F * N_SEG,), jnp.float32),
            jax.ShapeDtypeStruct((B * N_SEG,), jnp.float32),
            jax.ShapeDtypeStruct((B * 2 * N_SEG,), jnp.float32),
        ],
        scratch_types=[
            pltpu.VMEM((2, CHUNK), jnp.int32),
            pltpu.VMEM((2 * CPW * CHUNK,), jnp.float32),
            pltpu.VMEM((CPW * N_SEG,), jnp.float32),
            pltpu.VMEM((N_SEG,), jnp.float32),
            pltpu.VMEM((N_SEG,), jnp.float32),
            pltpu.VMEM((N_SEG,), jnp.float32),
            pltpu.VMEM((2 * N_SEG,), jnp.float32),
            pltpu.VMEM((2, N_SEG), jnp.float32),
            pltpu.VMEM_SHARED((2, F, N_SEG), jnp.float32),
            pltpu.VMEM_SHARED((2, 2, N_SEG), jnp.float32),
            pltpu.SemaphoreType.DMA,
            pltpu.SemaphoreType.DMA,
        ],
    )
    def sp_kernel(feat_hbm, seg_hbm, tok_hbm, cnt_hbm, cent_hbm,
                  seg_v, feat_v, acc_v, cnt_v, cent_v, recip_v,
                  cint_v, cpair_v, sp_stripes, sp_cent,
                  sem0, sem1):
        cid = lax.axis_index("c")
        sid = lax.axis_index("s")
        batch = cid * (B // 2) + sid // WPB
        wib = sid % WPB
        c0 = wib * CPW

        iota = lax.iota(jnp.int32, L)
        zeros = jnp.zeros((L,), jnp.float32)
        ones = jnp.ones((L,), jnp.float32)
        col_base = [jnp.full((L,), cc * N_SEG, jnp.int32) for cc in range(CPW)]

        def zero_small(i, _):
            cnt_v[pl.ds(i * L, L)] = zeros
            cent_v[pl.ds(i * L, L)] = zeros
            return 0
        lax.fori_loop(0, N_SEG // L, zero_small, 0)

        @plsc.parallel_loop(0, (CPW * N_SEG) // L, unroll=8)
        def zero_acc(i):
            acc_v[pl.ds(i * L, L)] = zeros

        feat_base = (batch * F + c0) * P
        sems = [sem0, sem1]

        def copies(g, sl):
            base_px = g * CHUNK
            cps = [pltpu.make_async_copy(
                seg_hbm.at[pl.ds(batch * P + base_px, CHUNK)],
                seg_v.at[sl], sems[sl])]
            for cc in range(CPW):
                cps.append(pltpu.make_async_copy(
                    feat_hbm.at[pl.ds(feat_base + cc * P + base_px, CHUNK)],
                    feat_v.at[pl.ds((sl * CPW + cc) * CHUNK, CHUNK)],
                    sems[sl]))
            return cps

        def issue(g, sl):
            for cp in copies(g, sl):
                cp.start()

        def drain(g, sl):
            for cp in copies(g, sl):
                cp.wait()

        issue(0, 0)
        issue(1, 1)

        def process(g, sl, base_px):
            @plsc.parallel_loop(0, GROUPS, unroll=4)
            def grp(i):
                off = i * L
                seg16 = seg_v[sl, pl.ds(off, L)]
                for cc in range(CPW):
                    val = feat_v[pl.ds((sl * CPW + cc) * CHUNK + off, L)]
                    plsc.addupdate_scatter(acc_v, [seg16 + col_base[cc]], val)
                plsc.addupdate_scatter(cnt_v, [seg16], ones)

            @pl.when(jnp.logical_or(wib == 1, wib == 2))
            def _():
                is_x = jnp.full((L,), wib) == 1

                @plsc.parallel_loop(0, GROUPS, unroll=4)
                def grpc(i):
                    off = i * L
                    seg16 = seg_v[sl, pl.ds(off, L)]
                    p = base_px + off + iota
                    x = (p & 511).astype(jnp.float32)
                    y = (p >> 9).astype(jnp.float32)
                    v = jnp.where(is_x, x, y) * (1.0 / 511.0)
                    plsc.addupdate_scatter(cent_v, [seg16], v)

        def outer(gg, _):
            for sl in range(2):
                g = gg * 2 + sl
                drain(g, sl)
                process(g, sl, g * CHUNK)

                @pl.when(g + 2 < NCHUNK)
                def _():
                    issue(g + 2, sl)
            return 0
        lax.fori_loop(0, NCHUNK // 2, outer, 0)

        def rec(i, _):
            c16 = cnt_v[pl.ds(i * L, L)]
            recip_v[pl.ds(i * L, L)] = 1.0 / jnp.maximum(c16, 1.0)
            return 0
        lax.fori_loop(0, N_SEG // L, rec, 0)

        @plsc.parallel_loop(0, (CPW * N_SEG) // L, unroll=8)
        def div_acc(i):
            s = pl.ds(i * L, L)
            r = pl.ds((i * L) & (N_SEG - 1), L)
            acc_v[s] = acc_v[s] * recip_v[r]

        bb = sid // WPB
        stripe_cps = [
            pltpu.make_async_copy(
                acc_v.at[pl.ds(cc * N_SEG, N_SEG)],
                sp_stripes.at[bb, c0 + cc], sem0)
            for cc in range(CPW)]
        for cp in stripe_cps:
            cp.start()

        @pl.when(jnp.logical_or(wib == 1, wib == 2))
        def _():
            def div_cent(i, _):
                s = pl.ds(i * L, L)
                cent_v[s] = cent_v[s] * recip_v[s]
                return 0
            lax.fori_loop(0, N_SEG // L, div_cent, 0)
            pltpu.sync_copy(cent_v, sp_cent.at[bb, wib - 1])

        for cp in stripe_cps:
            cp.wait()
        plsc.subcore_barrier()

        @pl.when(wib == 0)
        def _():
            pltpu.sync_copy(cnt_v, cnt_hbm.at[pl.ds(batch * N_SEG, N_SEG)])

        SEGW = N_SEG // WPB
        s0 = wib * SEGW
        read_cps = [
            pltpu.make_async_copy(
                sp_stripes.at[bb, c, pl.ds(s0, SEGW)],
                feat_v.at[pl.ds(c * SEGW, SEGW)], sem1)
            for c in range(F)]
        for cp in read_cps:
            cp.start()
        for cp in read_cps:
            cp.wait()

        @plsc.parallel_loop(0, (SEGW * F) // L, unroll=4)
        def transpose_grp(i):
            e16 = i * L + iota
            s16 = e16 // F
            c16 = e16 - s16 * F
            acc_v[pl.ds(i * L, L)] = plsc.load_gather(
                feat_v, [c16 * SEGW + s16])

        pltpu.sync_copy(
            acc_v.at[pl.ds(0, SEGW * F)],
            tok_hbm.at[pl.ds((batch * N_SEG + s0) * F, SEGW * F)])

        @pl.when(wib == 3)
        def _():
            pltpu.sync_copy(sp_cent.at[bb], cpair_v)

            @plsc.parallel_loop(0, (2 * N_SEG) // L, unroll=4)
            def cent_grp(i):
                e16 = i * L + iota
                s16 = e16 >> 1
                k16 = e16 & 1
                cint_v[pl.ds(i * L, L)] = plsc.load_gather(cpair_v,
                                                           [k16, s16])
            pltpu.sync_copy(
                cint_v, cent_hbm.at[pl.ds(batch * 2 * N_SEG, 2 * N_SEG)])

    return sp_kernel


def kernel(images, features, segment_map):
    B, F, H, W = features.shape
    P = H * W
    feats = features.reshape(B * F * P)
    segs = segment_map.reshape(B * P)
    tok1d, cnt1d, cent1d = _build_sc_kernel(B, F, P, 12, 2048)(feats, segs)
    tokens = tok1d.reshape(B, N_SEG, F)
    counts = cnt1d.reshape(B, N_SEG)
    centroids = cent1d.reshape(B, N_SEG, 2)
    attention_mask = counts > 0
    return (tokens, segment_map, attention_mask, centroids)


# speedup vs baseline: 1.1799x; 1.0228x over previous
import functools

import jax
import jax.numpy as jnp
from jax import lax
from jax.experimental import pallas as pl
from jax.experimental.pallas import tpu as pltpu
from jax.experimental.pallas import tpu_sc as plsc

N_SEG = 2048
L = 16


@functools.lru_cache(maxsize=None)
def _build_sc_kernel(B, F, P, CPW, CHUNK):
    NCHUNK = P // CHUNK
    GROUPS = CHUNK // L
    WPB = F // CPW
    assert B * WPB == 32, "mapping assumes 32 SC subcores"

    mesh = plsc.VectorSubcoreMesh(core_axis_name="c", subcore_axis_name="s")

    @functools.partial(
        pl.kernel,
        mesh=mesh,
        compiler_params=pltpu.CompilerParams(needs_layout_passes=False),
        out_type=[
            jax.ShapeDtypeStruct((B * F * N_SEG,), jnp.float32),
            jax.ShapeDtypeStruct((B * N_SEG,), jnp.float32),
            jax.ShapeDtypeStruct((B * 2 * N_SEG,), jnp.float32),
        ],
        scratch_types=[
            pltpu.VMEM((2, CHUNK), jnp.int32),
            pltpu.VMEM((2 * CPW * CHUNK,), jnp.float32),
            pltpu.VMEM((CPW * N_SEG,), jnp.float32),
            pltpu.VMEM((N_SEG,), jnp.float32),
            pltpu.VMEM((N_SEG,), jnp.float32),
            pltpu.SemaphoreType.DMA,
            pltpu.SemaphoreType.DMA,
        ],
    )
    def sp_kernel(feat_hbm, seg_hbm, sum_hbm, cnt_hbm, cent_hbm,
                  seg_v, feat_v, acc_v, cnt_v, cent_v, sem0, sem1):
        cid = lax.axis_index("c")
        sid = lax.axis_index("s")
        batch = cid * (B // 2) + sid // WPB
        wib = sid % WPB
        c0 = wib * CPW

        iota = lax.iota(jnp.int32, L)
        zeros = jnp.zeros((L,), jnp.float32)
        ones = jnp.ones((L,), jnp.float32)
        col_base = [jnp.full((L,), cc * N_SEG, jnp.int32) for cc in range(CPW)]

        def zero_small(i, _):
            cnt_v[pl.ds(i * L, L)] = zeros
            cent_v[pl.ds(i * L, L)] = zeros
            return 0
        lax.fori_loop(0, N_SEG // L, zero_small, 0)

        @plsc.parallel_loop(0, (CPW * N_SEG) // L, unroll=8)
        def zero_acc(i):
            acc_v[pl.ds(i * L, L)] = zeros

        feat_base = (batch * F + c0) * P
        sems = [sem0, sem1]

        def copies(g, sl):
            base_px = g * CHUNK
            cps = [pltpu.make_async_copy(
                seg_hbm.at[pl.ds(batch * P + base_px, CHUNK)],
                seg_v.at[sl], sems[sl])]
            for cc in range(CPW):
                cps.append(pltpu.make_async_copy(
                    feat_hbm.at[pl.ds(feat_base + cc * P + base_px, CHUNK)],
                    feat_v.at[pl.ds((sl * CPW + cc) * CHUNK, CHUNK)],
                    sems[sl]))
            return cps

        def issue(g, sl):
            for cp in copies(g, sl):
                cp.start()

        def drain(g, sl):
            for cp in copies(g, sl):
                cp.wait()

        issue(0, 0)
        issue(1, 1)

        def process(g, sl, base_px):
            @plsc.parallel_loop(0, GROUPS, unroll=4)
            def grp(i):
                off = i * L
                seg16 = seg_v[sl, pl.ds(off, L)]
                for cc in range(CPW):
                    val = feat_v[pl.ds((sl * CPW + cc) * CHUNK + off, L)]
                    plsc.addupdate_scatter(acc_v, [seg16 + col_base[cc]], val)
                plsc.addupdate_scatter(cnt_v, [seg16], ones)

            @pl.when(jnp.logical_or(wib == 1, wib == 2))
            def _():
                is_x = jnp.full((L,), wib) == 1

                @plsc.parallel_loop(0, GROUPS, unroll=4)
                def grpc(i):
                    off = i * L
                    seg16 = seg_v[sl, pl.ds(off, L)]
                    p = base_px + off + iota
                    x = (p & 511).astype(jnp.float32)
                    y = (p >> 9).astype(jnp.float32)
                    v = jnp.where(is_x, x, y) * (1.0 / 511.0)
                    plsc.addupdate_scatter(cent_v, [seg16], v)

        def outer(gg, _):
            for sl in range(2):
                g = gg * 2 + sl
                drain(g, sl)
                process(g, sl, g * CHUNK)

                @pl.when(g + 2 < NCHUNK)
                def _():
                    issue(g + 2, sl)
            return 0
        lax.fori_loop(0, NCHUNK // 2, outer, 0)

        pltpu.sync_copy(
            acc_v, sum_hbm.at[pl.ds((batch * F + c0) * N_SEG, CPW * N_SEG)])

        @pl.when(wib == 0)
        def _():
            pltpu.sync_copy(cnt_v, cnt_hbm.at[pl.ds(batch * N_SEG, N_SEG)])

        @pl.when(jnp.logical_or(wib == 1, wib == 2))
        def _():
            pltpu.sync_copy(
                cent_v,
                cent_hbm.at[pl.ds((batch * 2 + (wib - 1)) * N_SEG, N_SEG)])

    return sp_kernel


@functools.lru_cache(maxsize=None)
def _build_tc_finalize(B, F):
    def fin(sums_ref, cnt_ref, xy_ref, tok_ref, mask_ref, cent_ref):
        cnt = cnt_ref[...]
        recip = 1.0 / jnp.maximum(cnt, 1.0)
        means = sums_ref[...] * recip[None, :]
        tok_ref[0] = means.T
        mask_ref[0, 0] = cnt > 0
        xy = xy_ref[...].reshape(2, N_SEG) * recip[None, :]
        cent_ref[0] = xy.T

    return pl.pallas_call(
        fin,
        grid=(B,),
        in_specs=[
            pl.BlockSpec((F, N_SEG), lambda b: (b, 0)),
            pl.BlockSpec((N_SEG,), lambda b: (b,)),
            pl.BlockSpec((2 * N_SEG,), lambda b: (b,)),
        ],
        out_specs=[
            pl.BlockSpec((1, N_SEG, F), lambda b: (b, 0, 0)),
            pl.BlockSpec((1, 1, N_SEG), lambda b: (b, 0, 0)),
            pl.BlockSpec((1, N_SEG, 2), lambda b: (b, 0, 0)),
        ],
        out_shape=[
            jax.ShapeDtypeStruct((B, N_SEG, F), jnp.float32),
            jax.ShapeDtypeStruct((B, 1, N_SEG), jnp.bool_),
            jax.ShapeDtypeStruct((B, N_SEG, 2), jnp.float32),
        ],
    )


def kernel(images, features, segment_map):
    B, F, H, W = features.shape
    P = H * W
    feats = features.reshape(B * F * P)
    segs = segment_map.reshape(B * P)
    sum1d, cnt1d, xy1d = _build_sc_kernel(B, F, P, 12, 2048)(feats, segs)
    tokens, mask3d, centroids = _build_tc_finalize(B, F)(
        sum1d.reshape(B * F, N_SEG), cnt1d, xy1d)
    return (tokens, segment_map, mask3d.reshape(B, N_SEG), centroids)


# speedup vs baseline: 1.8812x; 1.5944x over previous
import functools

import jax
import jax.numpy as jnp
from jax import lax
from jax.experimental import pallas as pl
from jax.experimental.pallas import tpu as pltpu
from jax.experimental.pallas import tpu_sc as plsc

N_SEG = 2048
L = 16


@functools.lru_cache(maxsize=None)
def _build_sc_kernel(B, F, H, W, CPW):
    RPC = 8
    CHUNK = RPC * W
    NCHUNK = H // RPC
    CPU_ = CPW // 2
    GROUPS = CHUNK // L
    WPB = F // CPW
    assert B * WPB == 32, "mapping assumes 32 SC subcores"

    mesh = plsc.VectorSubcoreMesh(core_axis_name="c", subcore_axis_name="s")

    @functools.partial(
        pl.kernel,
        mesh=mesh,
        compiler_params=pltpu.CompilerParams(needs_layout_passes=False),
        out_type=[
            jax.ShapeDtypeStruct((B * F * N_SEG,), jnp.float32),
            jax.ShapeDtypeStruct((B * N_SEG,), jnp.float32),
            jax.ShapeDtypeStruct((B * 2 * N_SEG,), jnp.float32),
        ],
        scratch_types=[
            pltpu.VMEM((2, RPC, W), jnp.int32),
            pltpu.VMEM((2, CPU_, RPC, W), jnp.float32),
            pltpu.VMEM((2, RPC, W), jnp.float32),
            pltpu.VMEM((CPW * N_SEG,), jnp.float32),
            pltpu.VMEM((N_SEG,), jnp.float32),
            pltpu.VMEM((N_SEG,), jnp.float32),
            pltpu.SemaphoreType.DMA,
            pltpu.SemaphoreType.DMA,
            pltpu.SemaphoreType.DMA,
            pltpu.SemaphoreType.DMA,
        ],
    )
    def sp_kernel(feat_hbm, seg_hbm, coord_hbm, sum_hbm, cnt_hbm, cent_hbm,
                  seg_v, feat_v, coord_v, acc_v, cnt_v, cent_v,
                  fsem0, fsem1, ssem0, ssem1):
        cid = lax.axis_index("c")
        sid = lax.axis_index("s")
        batch = cid * (B // 2) + sid // WPB
        wib = sid % WPB
        c0 = wib * CPW
        is_cent = jnp.logical_or(wib == 1, wib == 2)

        iota = lax.iota(jnp.int32, L)
        zeros = jnp.zeros((L,), jnp.float32)
        ones = jnp.ones((L,), jnp.float32)
        col_base = [jnp.full((L,), cc * N_SEG, jnp.int32) for cc in range(CPW)]

        def zero_small(i, _):
            cnt_v[pl.ds(i * L, L)] = zeros
            cent_v[pl.ds(i * L, L)] = zeros
            return 0
        lax.fori_loop(0, N_SEG // L, zero_small, 0)

        @plsc.parallel_loop(0, (CPW * N_SEG) // L, unroll=8)
        def zero_acc(i):
            acc_v[pl.ds(i * L, L)] = zeros

        fsems = [fsem0, fsem1]
        ssems = [ssem0, ssem1]

        def feat_copies(g, half):
            r0 = g * RPC
            return [pltpu.make_async_copy(
                feat_hbm.at[batch, c0 + half * CPU_ + cc, pl.ds(r0, RPC), :],
                feat_v.at[half, cc], fsems[half])
                for cc in range(CPU_)]

        def seg_copies(g, sl):
            r0 = g * RPC
            return [pltpu.make_async_copy(
                seg_hbm.at[batch, pl.ds(r0, RPC), :], seg_v.at[sl],
                ssems[sl])]

        def coord_copies(g, sl):
            r0 = g * RPC
            return [pltpu.make_async_copy(
                coord_hbm.at[wib - 1, pl.ds(r0, RPC), :], coord_v.at[sl],
                ssems[sl])]

        def start(cps):
            for cp in cps:
                cp.start()

        def wait(cps):
            for cp in cps:
                cp.wait()

        start(seg_copies(0, 0))

        @pl.when(is_cent)
        def _():
            start(coord_copies(0, 0))
        start(feat_copies(0, 0))
        start(feat_copies(0, 1))

        def process_unit(g, half, sl):
            @plsc.parallel_loop(0, GROUPS, unroll=4)
            def grp(i):
                r = i >> 5
                cq = (i & 31) * L
                seg16 = seg_v[sl, r, pl.ds(cq, L)]
                for cc in range(CPU_):
                    val = feat_v[half, cc, r, pl.ds(cq, L)]
                    plsc.addupdate_scatter(
                        acc_v, [seg16 + col_base[half * CPU_ + cc]], val)

        def process_extras(g, sl):
            @pl.when(wib == 0)
            def _():
                @plsc.parallel_loop(0, GROUPS, unroll=4)
                def grpn(i):
                    r = i >> 5
                    cq = (i & 31) * L
                    seg16 = seg_v[sl, r, pl.ds(cq, L)]
                    plsc.addupdate_scatter(cnt_v, [seg16], ones)

            @pl.when(is_cent)
            def _():
                @plsc.parallel_loop(0, GROUPS, unroll=4)
                def grpc(i):
                    r = i >> 5
                    cq = (i & 31) * L
                    seg16 = seg_v[sl, r, pl.ds(cq, L)]
                    v = coord_v[sl, r, pl.ds(cq, L)]
                    plsc.addupdate_scatter(cent_v, [seg16], v)

        def outer(gg, _):
            for j in range(2):
                g = gg * 2 + j
                for half in range(2):
                    wait(feat_copies(g, half))
                    if half == 0:
                        wait(seg_copies(g, j))

                        @pl.when(jnp.logical_and(is_cent, g > 0))
                        def _():
                            wait(coord_copies(g, j))

                        @pl.when(g + 1 < NCHUNK)
                        def _():
                            start(seg_copies(g + 1, 1 - j))

                            @pl.when(is_cent)
                            def _():
                                start(coord_copies(g + 1, 1 - j))
                    process_unit(g, half, j)

                    @pl.when(g + 1 < NCHUNK)
                    def _():
                        start(feat_copies(g + 1, half))
                process_extras(g, j)
            return 0
        lax.fori_loop(0, NCHUNK // 2, outer, 0)

        pltpu.sync_copy(
            acc_v, sum_hbm.at[pl.ds((batch * F + c0) * N_SEG, CPW * N_SEG)])

        @pl.when(wib == 0)
        def _():
            pltpu.sync_copy(cnt_v, cnt_hbm.at[pl.ds(batch * N_SEG, N_SEG)])

        @pl.when(is_cent)
        def _():
            pltpu.sync_copy(
                cent_v,
                cent_hbm.at[pl.ds((batch * 2 + (wib - 1)) * N_SEG, N_SEG)])

    return sp_kernel


@functools.lru_cache(maxsize=None)
def _build_tc_finalize(B, F):
    def fin(sums_ref, cnt_ref, xy_ref, tok_ref, mask_ref, cent_ref):
        cnt = cnt_ref[...]
        recip = 1.0 / jnp.maximum(cnt, 1.0)
        means = sums_ref[...] * recip[None, :]
        tok_ref[0] = means.T
        mask_ref[0, 0] = cnt > 0
        xy = xy_ref[...].reshape(2, N_SEG) * recip[None, :]
        cent_ref[0] = xy.T

    return pl.pallas_call(
        fin,
        grid=(B,),
        in_specs=[
            pl.BlockSpec((F, N_SEG), lambda b: (b, 0)),
            pl.BlockSpec((N_SEG,), lambda b: (b,)),
            pl.BlockSpec((2 * N_SEG,), lambda b: (b,)),
        ],
        out_specs=[
            pl.BlockSpec((1, N_SEG, F), lambda b: (b, 0, 0)),
            pl.BlockSpec((1, 1, N_SEG), lambda b: (b, 0, 0)),
            pl.BlockSpec((1, N_SEG, 2), lambda b: (b, 0, 0)),
        ],
        out_shape=[
            jax.ShapeDtypeStruct((B, N_SEG, F), jnp.float32),
            jax.ShapeDtypeStruct((B, 1, N_SEG), jnp.bool_),
            jax.ShapeDtypeStruct((B, N_SEG, 2), jnp.float32),
        ],
    )


def kernel(images, features, segment_map):
    B, F, H, W = features.shape
    x = jnp.arange(W, dtype=jnp.float32) / (W - 1)
    y = jnp.arange(H, dtype=jnp.float32) / (H - 1)
    coords = jnp.stack([
        jnp.broadcast_to(x[None, :], (H, W)),
        jnp.broadcast_to(y[:, None], (H, W)),
    ])
    sum1d, cnt1d, xy1d = _build_sc_kernel(B, F, H, W, 12)(
        features, segment_map, coords)
    tokens, mask3d, centroids = _build_tc_finalize(B, F)(
        sum1d.reshape(B * F, N_SEG), cnt1d, xy1d)
    return (tokens, segment_map, mask3d.reshape(B, N_SEG), centroids)
